# baseline (speedup 1.0000x reference)
import functools

import jax
import jax.numpy as jnp
from jax import lax
from jax.experimental import pallas as pl
from jax.experimental.pallas import tpu as pltpu
from jax.experimental.pallas import tpu_sc as plsc

_LANES = 16
_CHUNK = 128


def _sc_gather_add(table1, table2, idx1, idx2, n_rows, embed_dim):
    info = plsc.get_sparse_core_info()
    nc, ns = info.num_cores, info.num_subcores
    nw = nc * ns
    per_w = n_rows // nw
    n_chunks = per_w // _CHUNK

    mesh = plsc.VectorSubcoreMesh(core_axis_name="c", subcore_axis_name="s")

    @functools.partial(
        pl.kernel,
        mesh=mesh,
        out_type=jax.ShapeDtypeStruct((n_rows, embed_dim), jnp.float32),
        scratch_types=[
            pltpu.VMEM((n_chunks, _CHUNK), jnp.int32),
            pltpu.VMEM((n_chunks, _CHUNK), jnp.int32),
            pltpu.VMEM((_CHUNK, embed_dim), jnp.float32),
            pltpu.VMEM((_CHUNK, embed_dim), jnp.float32),
            pltpu.SemaphoreType.DMA,
            pltpu.SemaphoreType.DMA,
        ],
        compiler_params=pltpu.CompilerParams(use_tc_tiling_on_sc=False),
    )
    def gather_kernel(t1, t2, i1, i2, out, i1_v, i2_v, r1, r2, s1, s2):
        wid = lax.axis_index("s") * nc + lax.axis_index("c")
        pltpu.sync_copy(i1.at[wid], i1_v)
        pltpu.sync_copy(i2.at[wid], i2_v)
        base = wid * per_w

        def chunk(g, carry):
            c1 = pltpu.async_copy(t1.at[i1_v.at[g]], r1, s1)
            c2 = pltpu.async_copy(t2.at[i2_v.at[g]], r2, s2)
            c1.wait()
            c2.wait()

            def add_row(i, c):
                for j in range(embed_dim // _LANES):
                    sl = (i, pl.ds(j * _LANES, _LANES))
                    r1[sl] = r1[sl] + r2[sl]
                return c

            lax.fori_loop(0, _CHUNK, add_row, 0)
            pltpu.sync_copy(r1, out.at[pl.ds(base + g * _CHUNK, _CHUNK)])
            return carry

        lax.fori_loop(0, n_chunks, chunk, 0)

    return gather_kernel(table1, table2, idx1, idx2)


def _tc_mlp(emb3, W1, W2, b2):
    L, B, D = emb3.shape
    H = W1.shape[0]
    blk = min(B, 2048)
    nb = B // blk
    b2m = b2.reshape(1, 1)

    def body(e_ref, w1_ref, w2_ref, b2_ref, o_ref):
        l = pl.program_id(1)
        x = e_ref[0]
        h = lax.dot_general(
            x, w1_ref[...], (((1,), (1,)), ((), ())),
            preferred_element_type=jnp.float32)
        h = jnp.maximum(h, 0.0)
        y = jnp.sum(h * w2_ref[...], axis=1, keepdims=True)

        @pl.when(l == 0)
        def _init():
            o_ref[...] = jnp.zeros_like(o_ref)

        o_ref[...] += y + b2_ref[0, 0]

    return pl.pallas_call(
        body,
        grid=(nb, L),
        in_specs=[
            pl.BlockSpec((1, blk, D), lambda i, l: (l, i, 0)),
            pl.BlockSpec((H, D), lambda i, l: (0, 0)),
            pl.BlockSpec((1, H), lambda i, l: (0, 0)),
            pl.BlockSpec((1, 1), lambda i, l: (0, 0)),
        ],
        out_specs=pl.BlockSpec((blk, 1), lambda i, l: (i, 0)),
        out_shape=jax.ShapeDtypeStruct((B, 1), jnp.float32),
    )(emb3, W1, W2, b2m)


def kernel(input1, input2, table1, table2, W1, W2, b2):
    B, L = input1.shape
    D = table1.shape[1]
    n_rows = B * L

    info = plsc.get_sparse_core_info()
    nw = info.num_cores * info.num_subcores
    n_chunks = n_rows // (nw * _CHUNK)

    idx1 = input1.astype(jnp.int32).T.reshape(nw, n_chunks, _CHUNK)
    idx2 = input2.astype(jnp.int32).T.reshape(nw, n_chunks, _CHUNK)

    emb = _sc_gather_add(table1, table2, idx1, idx2, n_rows, D)
    return _tc_mlp(emb.reshape(L, B, D), W1, W2, b2)


# speedup vs baseline: 1.0133x; 1.0133x over previous
import functools

import jax
import jax.numpy as jnp
from jax import lax
from jax.experimental import pallas as pl
from jax.experimental.pallas import tpu as pltpu
from jax.experimental.pallas import tpu_sc as plsc

_LANES = 16
_CHUNK = 128


def _sc_gather_add(table1, table2, idx1, idx2, n_rows, embed_dim):
    info = plsc.get_sparse_core_info()
    nc, ns = info.num_cores, info.num_subcores
    nw = nc * ns
    per_w = n_rows // nw
    n_chunks = per_w // _CHUNK

    mesh = plsc.VectorSubcoreMesh(core_axis_name="c", subcore_axis_name="s")

    @functools.partial(
        pl.kernel,
        mesh=mesh,
        out_type=jax.ShapeDtypeStruct((n_rows, embed_dim), jnp.float32),
        scratch_types=[
            pltpu.VMEM((n_chunks, _CHUNK), jnp.int32),
            pltpu.VMEM((n_chunks, _CHUNK), jnp.int32),
            pltpu.VMEM((_CHUNK, embed_dim), jnp.float32),
            pltpu.VMEM((_CHUNK, embed_dim), jnp.float32),
            pltpu.SemaphoreType.DMA,
            pltpu.SemaphoreType.DMA,
        ],
        compiler_params=pltpu.CompilerParams(use_tc_tiling_on_sc=False),
    )
    def gather_kernel(t1, t2, i1, i2, out, i1_v, i2_v, r1, r2, s1, s2):
        wid = lax.axis_index("s") * nc + lax.axis_index("c")
        pltpu.sync_copy(i1.at[wid], i1_v)
        pltpu.sync_copy(i2.at[wid], i2_v)
        base = wid * per_w

        def chunk(g, carry):
            c1 = pltpu.async_copy(t1.at[i1_v.at[g]], r1, s1)
            c2 = pltpu.async_copy(t2.at[i2_v.at[g]], r2, s2)
            c1.wait()
            c2.wait()

            def add_row(i, c):
                for j in range(embed_dim // _LANES):
                    sl = (i, pl.ds(j * _LANES, _LANES))
                    r1[sl] = r1[sl] + r2[sl]
                return c

            lax.fori_loop(0, _CHUNK, add_row, 0)
            pltpu.sync_copy(r1, out.at[pl.ds(base + g * _CHUNK, _CHUNK)])
            return carry

        lax.fori_loop(0, n_chunks, chunk, 0)

    return gather_kernel(table1, table2, idx1, idx2)


def _tc_mlp(emb3, W1, W2, b2):
    L, B, D = emb3.shape
    H = W1.shape[0]
    blk = min(B, 2048)
    nb = B // blk
    w2col = W2.reshape(H, 1)
    b2m = b2.reshape(1, 1)

    def body(e_ref, w1_ref, w2_ref, b2_ref, o_ref):
        l = pl.program_id(1)
        x = e_ref[0]
        ht = lax.dot_general(
            w1_ref[...], x, (((1,), (1,)), ((), ())),
            preferred_element_type=jnp.float32)
        ht = jnp.maximum(ht, 0.0)
        y = jnp.sum(ht * w2_ref[...], axis=0, keepdims=True)

        @pl.when(l == 0)
        def _init():
            o_ref[...] = jnp.zeros_like(o_ref)

        o_ref[...] += y + b2_ref[0, 0]

    out_row = pl.pallas_call(
        body,
        grid=(nb, L),
        in_specs=[
            pl.BlockSpec((1, blk, D), lambda i, l: (l, i, 0)),
            pl.BlockSpec((H, D), lambda i, l: (0, 0)),
            pl.BlockSpec((H, 1), lambda i, l: (0, 0)),
            pl.BlockSpec((1, 1), lambda i, l: (0, 0)),
        ],
        out_specs=pl.BlockSpec((1, blk), lambda i, l: (0, i)),
        out_shape=jax.ShapeDtypeStruct((1, B), jnp.float32),
    )(emb3, W1, w2col, b2m)
    return out_row.reshape(B, 1)


def kernel(input1, input2, table1, table2, W1, W2, b2):
    B, L = input1.shape
    D = table1.shape[1]
    n_rows = B * L

    info = plsc.get_sparse_core_info()
    nw = info.num_cores * info.num_subcores
    n_chunks = n_rows // (nw * _CHUNK)

    idx1 = input1.astype(jnp.int32).T.reshape(nw, n_chunks, _CHUNK)
    idx2 = input2.astype(jnp.int32).T.reshape(nw, n_chunks, _CHUNK)

    emb = _sc_gather_add(table1, table2, idx1, idx2, n_rows, D)
    return _tc_mlp(emb.reshape(L, B, D), W1, W2, b2)


# speedup vs baseline: 1.0498x; 1.0359x over previous
import functools

import jax
import jax.numpy as jnp
from jax import lax
from jax.experimental import pallas as pl
from jax.experimental.pallas import tpu as pltpu
from jax.experimental.pallas import tpu_sc as plsc

_LANES = 16
_CHUNK = 128


def _sc_gather_add(table1, table2, idx1f, idx2f):
    n_rows = idx1f.shape[0]
    embed_dim = table1.shape[1]
    info = plsc.get_sparse_core_info()
    nc, ns = info.num_cores, info.num_subcores
    nw = nc * ns
    per_w = n_rows // nw
    n_chunks = per_w // _CHUNK

    mesh = plsc.VectorSubcoreMesh(core_axis_name="c", subcore_axis_name="s")

    @functools.partial(
        pl.kernel,
        mesh=mesh,
        out_type=jax.ShapeDtypeStruct((n_rows, embed_dim), jnp.float32),
        scratch_types=[
            pltpu.VMEM((per_w,), jnp.int32),
            pltpu.VMEM((per_w,), jnp.int32),
            pltpu.VMEM((_CHUNK, embed_dim), jnp.float32),
            pltpu.VMEM((_CHUNK, embed_dim), jnp.float32),
            pltpu.SemaphoreType.DMA,
            pltpu.SemaphoreType.DMA,
        ],
        compiler_params=pltpu.CompilerParams(use_tc_tiling_on_sc=False),
    )
    def gather_kernel(t1, t2, i1, i2, out, i1_v, i2_v, r1, r2, s1, s2):
        wid = lax.axis_index("s") * nc + lax.axis_index("c")
        base = wid * per_w
        pltpu.sync_copy(i1.at[pl.ds(base, per_w)], i1_v)
        pltpu.sync_copy(i2.at[pl.ds(base, per_w)], i2_v)

        def chunk(g, carry):
            c1 = pltpu.async_copy(
                t1.at[i1_v.at[pl.ds(g * _CHUNK, _CHUNK)]], r1, s1)
            c2 = pltpu.async_copy(
                t2.at[i2_v.at[pl.ds(g * _CHUNK, _CHUNK)]], r2, s2)
            c1.wait()
            c2.wait()

            def add_row(i, c):
                for j in range(embed_dim // _LANES):
                    sl = (i, pl.ds(j * _LANES, _LANES))
                    r1[sl] = r1[sl] + r2[sl]
                return c

            lax.fori_loop(0, _CHUNK, add_row, 0)
            pltpu.sync_copy(r1, out.at[pl.ds(base + g * _CHUNK, _CHUNK)])
            return carry

        lax.fori_loop(0, n_chunks, chunk, 0)

    return gather_kernel(table1, table2, idx1f, idx2f)


def _tc_mlp(emb, W1, W2, b2, B, L):
    n_rows, D = emb.shape
    H = W1.shape[0]
    blk_b = 128
    blk_r = blk_b * L
    nb = B // blk_b
    w2col = W2.reshape(H, 1)
    b2m = b2.reshape(1, 1)
    gmat = (jnp.arange(blk_r, dtype=jnp.int32)[:, None] // L
            == jnp.arange(blk_b, dtype=jnp.int32)[None, :]
            ).astype(jnp.float32)

    def body(e_ref, w1_ref, w2_ref, b2_ref, g_ref, o_ref):
        x = e_ref[...]
        ht = lax.dot_general(
            w1_ref[...], x, (((1,), (1,)), ((), ())),
            preferred_element_type=jnp.float32)
        ht = jnp.maximum(ht, 0.0)
        y = jnp.sum(ht * w2_ref[...], axis=0, keepdims=True)
        o = lax.dot_general(
            y, g_ref[...], (((1,), (0,)), ((), ())),
            preferred_element_type=jnp.float32)
        o_ref[...] = o + L * b2_ref[0, 0]

    out_row = pl.pallas_call(
        body,
        grid=(nb,),
        in_specs=[
            pl.BlockSpec((blk_r, D), lambda i: (i, 0)),
            pl.BlockSpec((H, D), lambda i: (0, 0)),
            pl.BlockSpec((H, 1), lambda i: (0, 0)),
            pl.BlockSpec((1, 1), lambda i: (0, 0)),
            pl.BlockSpec((blk_r, blk_b), lambda i: (0, 0)),
        ],
        out_specs=pl.BlockSpec((1, blk_b), lambda i: (0, i)),
        out_shape=jax.ShapeDtypeStruct((1, B), jnp.float32),
    )(emb, W1, w2col, b2m, gmat)
    return out_row.reshape(B, 1)


def kernel(input1, input2, table1, table2, W1, W2, b2):
    B, L = input1.shape
    n_rows = B * L

    idx1f = input1.astype(jnp.int32).reshape(n_rows)
    idx2f = input2.astype(jnp.int32).reshape(n_rows)

    emb = _sc_gather_add(table1, table2, idx1f, idx2f)
    return _tc_mlp(emb, W1, W2, b2, B, L)
